# baseline (speedup 1.0000x reference)
import functools
import jax
import jax.numpy as jnp
from jax import lax
from jax.experimental import pallas as pl
from jax.experimental.pallas import tpu as pltpu
from jax.experimental.pallas import tpu_sc as plsc

_NU, _NI, _NT = 3000, 5000, 2000
_NN = _NU + _NI + _NT
_D = 128
_E = 320000

_NB = 64
_BS = 160
_NP = _NB * _BS
_F = 2048
_ECAP = 158 * _F
_BLK = 2000
_OCAP = 2 * _F
_G = 128
_NG = _G // 16
_NW = 32


def _mlp_body(emb_ref, w1_ref, b1_ref, w2_ref, b2_ref, out_ref):
    h = jnp.maximum(
        jnp.dot(emb_ref[...], w1_ref[...].T, preferred_element_type=jnp.float32)
        + b1_ref[...],
        0.0,
    )
    out_ref[...] = (
        jnp.dot(h, w2_ref[...].T, preferred_element_type=jnp.float32) + b2_ref[...]
    )


def _node_mlp(embs, W1, b1, W2, b2):
    blk = 1000
    return pl.pallas_call(
        _mlp_body,
        grid=(_NN // blk,),
        in_specs=[
            pl.BlockSpec((blk, _D), lambda i: (i, 0)),
            pl.BlockSpec((_D, _D), lambda i: (0, 0)),
            pl.BlockSpec((1, _D), lambda i: (0, 0)),
            pl.BlockSpec((_D, _D), lambda i: (0, 0)),
            pl.BlockSpec((1, _D), lambda i: (0, 0)),
        ],
        out_specs=pl.BlockSpec((blk, _D), lambda i: (i, 0)),
        out_shape=jax.ShapeDtypeStruct((_NN, _D), jnp.float32),
    )(embs, W1, b1.reshape(1, _D), W2, b2.reshape(1, _D))


def _finalize_body(num_ref, s_ref, offa_ref, offb_ref, emb_ref, off_ref):
    i = pl.program_id(0)
    agg = num_ref[...] / (s_ref[...] + 1e-16)
    nrm = jnp.sqrt(jnp.sum(agg * agg, axis=1, keepdims=True))
    emb_ref[...] = agg / jnp.maximum(nrm, 1e-12)
    blk = num_ref.shape[0]
    r = i * blk + lax.broadcasted_iota(jnp.int32, (blk, _D), 0)
    a = offa_ref[...]
    b = offb_ref[...]
    a0 = jnp.where(jnp.isfinite(a), a, 0.0)
    b0 = jnp.where(jnp.isfinite(b), b, 0.0)
    off = jnp.where(
        r < _NU, jnp.minimum(a0, b0), jnp.where(r < _NU + _NI, b0, a0)
    )
    off_ref[...] = jnp.maximum(off, 0.0)


def _finalize(num, s, offa, offb):
    blk = 1000
    return pl.pallas_call(
        _finalize_body,
        grid=(_NN // blk,),
        in_specs=[pl.BlockSpec((blk, _D), lambda i: (i, 0))] * 4,
        out_specs=[pl.BlockSpec((blk, _D), lambda i: (i, 0))] * 2,
        out_shape=[jax.ShapeDtypeStruct((_NN, _D), jnp.float32)] * 2,
    )(num, s, offa, offb)


_sc_mesh = plsc.VectorSubcoreMesh(core_axis_name="c", subcore_axis_name="s")


def _partition_body(head_hbm, tail_hbm, bp_hbm, cnts_hbm,
                    hblk, tblk, hblk2, tblk2, ot0, ot1, cnt16,
                    semA, semB, semf):
    wid = lax.axis_index("s") * 2 + lax.axis_index("c")
    b0 = wid
    b1 = wid + _NW
    lo0 = b0 * _BS
    hi0 = lo0 + _BS
    lo1 = b1 * _BS
    hi1 = lo1 + _BS

    lanes = lax.iota(jnp.int32, 16)
    bsets = ((hblk, tblk, semA), (hblk2, tblk2, semB))
    nblk = _E // _BLK

    def start_blk(i, bs):
        hb, tb, sm = bs
        pltpu.async_copy(
            head_hbm.at[pl.ds(pl.multiple_of(i * _BLK, 8), _BLK)], hb, sm
        )
        pltpu.async_copy(
            tail_hbm.at[pl.ds(pl.multiple_of(i * _BLK, 8), _BLK)], tb, sm
        )

    def wait_blk(bs):
        hb, tb, sm = bs
        pltpu.make_async_copy(head_hbm.at[pl.ds(0, _BLK)], hb, sm).wait()
        pltpu.make_async_copy(tail_hbm.at[pl.ds(0, _BLK)], tb, sm).wait()

    def blk_body(i, carry, hblk, tblk):
        w0, fl0, w1, fl1 = carry

        def vr_body(k, c2):
            w0, w1 = c2
            h = hblk[pl.ds(k * 16, 16)]
            t = tblk[pl.ds(k * 16, 16)]
            m0 = (h >= lo0) & (h < hi0)
            inc0 = plsc.cumsum(m0.astype(jnp.int32))
            pos0 = jnp.where(m0, w0 + inc0 - 1, _OCAP + lanes)
            plsc.store_scatter(ot0, [pos0], t | ((h - lo0) << 16))
            w0 = w0 + inc0[15]
            m1 = (h >= lo1) & (h < hi1)
            inc1 = plsc.cumsum(m1.astype(jnp.int32))
            pos1 = jnp.where(m1, w1 + inc1 - 1, _OCAP + lanes)
            plsc.store_scatter(ot1, [pos1], t | ((h - lo1) << 16))
            w1 = w1 + inc1[15]
            return (w0, w1)

        w0, w1 = lax.fori_loop(0, _BLK // 16, vr_body, (w0, w1))

        do0 = w0 >= _F

        @pl.when(do0)
        def _():
            pltpu.sync_copy(ot0.at[pl.ds(0, _F)],
                            bp_hbm.at[pl.ds(pl.multiple_of(b0 * _ECAP + fl0, 8), _F)])
            for k in range(_BLK // 16):
                ot0[pl.ds(k * 16, 16)] = ot0[pl.ds(_F + k * 16, 16)]

        w0 = jnp.where(do0, w0 - _F, w0)
        fl0 = jnp.where(do0, fl0 + _F, fl0)

        do1 = w1 >= _F

        @pl.when(do1)
        def _():
            pltpu.sync_copy(ot1.at[pl.ds(0, _F)],
                            bp_hbm.at[pl.ds(pl.multiple_of(b1 * _ECAP + fl1, 8), _F)])
            for k in range(_BLK // 16):
                ot1[pl.ds(k * 16, 16)] = ot1[pl.ds(_F + k * 16, 16)]

        w1 = jnp.where(do1, w1 - _F, w1)
        fl1 = jnp.where(do1, fl1 + _F, fl1)
        return (w0, fl0, w1, fl1)

    z = jnp.int32(0)
    for s in range(2):
        start_blk(jnp.int32(s), bsets[s])

    def pair(p, carry):
        for s in range(2):
            i = p * 2 + s
            bs = bsets[s]
            wait_blk(bs)
            carry = blk_body(i, carry, bs[0], bs[1])

            @pl.when(i + 2 < nblk)
            def _(i=i, bs=bs):
                start_blk(i + 2, bs)

        return carry

    w0, fl0, w1, fl1 = lax.fori_loop(0, nblk // 2, pair, (z, z, z, z))

    @pl.when(w0 > 0)
    def _():
        pltpu.sync_copy(ot0.at[pl.ds(0, _F)],
                        bp_hbm.at[pl.ds(pl.multiple_of(b0 * _ECAP + fl0, 8), _F)])

    cnt16[...] = jnp.full((16,), w0 + fl0, jnp.int32)
    pltpu.sync_copy(cnt16, cnts_hbm.at[pl.ds(pl.multiple_of(b0 * 16, 8), 16)])

    @pl.when(w1 > 0)
    def _():
        pltpu.sync_copy(ot1.at[pl.ds(0, _F)],
                        bp_hbm.at[pl.ds(pl.multiple_of(b1 * _ECAP + fl1, 8), _F)])

    cnt16[...] = jnp.full((16,), w1 + fl1, jnp.int32)
    pltpu.sync_copy(cnt16, cnts_hbm.at[pl.ds(pl.multiple_of(b1 * 16, 8), 16)])


_partition = functools.partial(
    pl.kernel,
    out_type=[
        jax.ShapeDtypeStruct((_NB * _ECAP,), jnp.int32),
        jax.ShapeDtypeStruct((_NB * 16,), jnp.int32),
    ],
    mesh=_sc_mesh,
    scratch_types=[
        pltpu.VMEM((_BLK,), jnp.int32),
        pltpu.VMEM((_BLK,), jnp.int32),
        pltpu.VMEM((_BLK,), jnp.int32),
        pltpu.VMEM((_BLK,), jnp.int32),
        pltpu.VMEM((_OCAP + 16,), jnp.int32),
        pltpu.VMEM((_OCAP + 16,), jnp.int32),
        pltpu.VMEM((16,), jnp.int32),
        pltpu.SemaphoreType.DMA,
        pltpu.SemaphoreType.DMA,
        pltpu.SemaphoreType.DMA,
    ],
    compiler_params=pltpu.CompilerParams(needs_layout_passes=False),
)(_partition_body)


def _hop_body(l2_hbm, off_hbm, emb_hbm, bp_hbm, cnts_hbm,
              offa_hbm, offb_hbm, s_hbm, num_hbm,
              acc0, acc1, acc2,
              pk0, pk1, ti0, ti1, hb0, hb1, r0a, r0b, r1a, r1b, counts_v,
              sem0, sem1, semi0, semi1):
    wid = lax.axis_index("s") * 2 + lax.axis_index("c")
    pltpu.sync_copy(cnts_hbm, counts_v)

    ninf = jnp.full((16,), -jnp.inf, jnp.float32)
    pinf = jnp.full((16,), jnp.inf, jnp.float32)
    zero = jnp.zeros((16,), jnp.float32)
    sets = (
        (pk0, ti0, hb0, r0a, r1a, sem0, semi0),
        (pk1, ti1, hb1, r0b, r1b, sem1, semi1),
    )

    def run_pass(bsafe, cnt, other_hbm, group_fn):
        nch = lax.div(cnt + (_G - 1), _G)

        def start_idx(c, st):
            pk = st[0]
            base = pl.multiple_of(bsafe * _ECAP + c * _G, 8)
            pltpu.async_copy(bp_hbm.at[pl.ds(base, _G)], pk, st[6])

        def wait_idx(st):
            pltpu.make_async_copy(bp_hbm.at[pl.ds(0, _G)], st[0], st[6]).wait()

        def stage_rows(st):
            pk, ti, hb, rr0, rr1, sm, smi = st
            for k in range(_NG):
                pv = pk[pl.ds(k * 16, 16)]
                ti[pl.ds(k * 16, 16)] = jnp.clip(pv & 0xFFFF, 0, _NN - 1)
                hb[pl.ds(k * 16, 16)] = jnp.clip(
                    lax.shift_right_arithmetic(pv, 16), 0, _BS - 1
                )
            pltpu.async_copy(l2_hbm.at[ti], rr0, sm)
            pltpu.async_copy(other_hbm.at[ti], rr1, sm)

        def wait_rows(st):
            pk, ti, hb, rr0, rr1, sm, smi = st
            pltpu.make_async_copy(l2_hbm.at[ti], rr0, sm).wait()
            pltpu.make_async_copy(other_hbm.at[ti], rr1, sm).wait()

        for s in range(2):
            @pl.when(s < nch)
            def _(s=s):
                st = sets[s]
                start_idx(jnp.int32(s), st)
                wait_idx(st)
                stage_rows(st)

                @pl.when(s + 2 < nch)
                def _(s=s, st=st):
                    start_idx(jnp.int32(s + 2), st)

        def pair(p, _):
            for s in range(2):
                c = p * 2 + s
                st = sets[s]

                @pl.when(c < nch)
                def _(c=c, st=st):
                    wait_rows(st)
                    nj = jnp.minimum(_G, cnt - c * _G)
                    group_fn(st, nj)

                    @pl.when(c + 2 < nch)
                    def _(c=c, st=st):
                        wait_idx(st)
                        stage_rows(st)

                        @pl.when(c + 4 < nch)
                        def _(c=c, st=st):
                            start_idx(c + 4, st)

            return 0

        lax.fori_loop(0, (nch + 1) // 2, pair, 0)

    def bucket_body(it, _):
        b = wid + it * _NW
        nb = b * _BS
        cnt = counts_v[pl.ds(b * 16, 16)][0]

        def init_a(k, _):
            acc0[pl.ds(k * 16, 16)] = ninf
            acc1[pl.ds(k * 16, 16)] = pinf
            acc2[pl.ds(k * 16, 16)] = ninf
            return 0

        lax.fori_loop(0, _BS * _D // 16, init_a, 0)

        def group_a(st, nj):
            pk, ti, hb, rr0, rr1, sm, smi = st

            def g_body(g, _):
                hv = hb[pl.ds(g * 16, 16)]
                tv = ti[pl.ds(g * 16, 16)]
                hs = [hv[l] for l in range(16)]
                ts = [tv[l] for l in range(16)]
                for l in range(16):
                    h = hs[l]
                    t = ts[l]
                    j = g * 16 + l
                    valid = j < nj
                    hg = h + nb
                    ca = ((hg < _NU) & (t >= _NU) & (t < _NU + _NI)) | (
                        hg >= _NU + _NI
                    )
                    cb = ((hg < _NU) & (t >= _NU + _NI)) | (
                        (hg >= _NU) & (hg < _NU + _NI)
                    )
                    sa = ca & valid
                    sb = cb & valid
                    rbase = h * _D
                    ld = []
                    for cc in range(8):
                        o = cc * 16
                        ld.append((
                            o,
                            rr0[j, pl.ds(o, 16)],
                            rr1[j, pl.ds(o, 16)],
                            acc0[pl.ds(rbase + o, 16)],
                            acc1[pl.ds(rbase + o, 16)],
                            acc2[pl.ds(rbase + o, 16)],
                        ))
                    for o, r0v, r1v, a0v, a1v, a2v in ld:
                        r1r = jnp.maximum(r1v, 0.0)
                        acc0[pl.ds(rbase + o, 16)] = jnp.maximum(
                            a0v, jnp.where(valid, r0v, ninf)
                        )
                        acc1[pl.ds(rbase + o, 16)] = jnp.minimum(
                            a1v, jnp.where(sa, r1r, pinf)
                        )
                        acc2[pl.ds(rbase + o, 16)] = jnp.maximum(
                            a2v, jnp.where(sb, r1r, ninf)
                        )
                return 0

            lax.fori_loop(0, _NG, g_body, 0)

        run_pass(b, cnt, off_hbm, group_a)

        pltpu.sync_copy(acc1, offa_hbm.at[pl.ds(pl.multiple_of(nb * _D, 8), _BS * _D)])
        pltpu.sync_copy(acc2, offb_hbm.at[pl.ds(pl.multiple_of(nb * _D, 8), _BS * _D)])

        def init_b(k, _):
            acc1[pl.ds(k * 16, 16)] = zero
            acc2[pl.ds(k * 16, 16)] = zero
            return 0

        lax.fori_loop(0, _BS * _D // 16, init_b, 0)

        def group_b(st, nj):
            pk, ti, hb, rr0, rr1, sm, smi = st

            def g_body(g, _):
                hv = hb[pl.ds(g * 16, 16)]
                hs = [hv[l] for l in range(16)]
                for l in range(16):
                    h = hs[l]
                    j = g * 16 + l
                    valid = j < nj
                    rbase = h * _D
                    ld = []
                    for cc in range(8):
                        o = cc * 16
                        ld.append((
                            o,
                            rr0[j, pl.ds(o, 16)],
                            rr1[j, pl.ds(o, 16)],
                            acc0[pl.ds(rbase + o, 16)],
                            acc1[pl.ds(rbase + o, 16)],
                            acc2[pl.ds(rbase + o, 16)],
                        ))
                    for o, l2v, embv, mv, s1v, n1v in ld:
                        ev = jnp.where(valid, jnp.exp(l2v - mv), zero)
                        acc1[pl.ds(rbase + o, 16)] = s1v + ev
                        acc2[pl.ds(rbase + o, 16)] = n1v + ev * embv
                return 0

            lax.fori_loop(0, _NG, g_body, 0)

        run_pass(b, cnt, emb_hbm, group_b)

        pltpu.sync_copy(acc1, s_hbm.at[pl.ds(pl.multiple_of(nb * _D, 8), _BS * _D)])
        pltpu.sync_copy(acc2, num_hbm.at[pl.ds(pl.multiple_of(nb * _D, 8), _BS * _D)])
        return 0

    lax.fori_loop(0, 2, bucket_body, 0)


_hop = functools.partial(
    pl.kernel,
    out_type=[jax.ShapeDtypeStruct((_NP * _D,), jnp.float32)] * 4,
    mesh=_sc_mesh,
    scratch_types=[
        pltpu.VMEM((_BS * _D,), jnp.float32),
        pltpu.VMEM((_BS * _D,), jnp.float32),
        pltpu.VMEM((_BS * _D,), jnp.float32),
        pltpu.VMEM((_G,), jnp.int32),
        pltpu.VMEM((_G,), jnp.int32),
        pltpu.VMEM((_G,), jnp.int32),
        pltpu.VMEM((_G,), jnp.int32),
        pltpu.VMEM((_G,), jnp.int32),
        pltpu.VMEM((_G,), jnp.int32),
        pltpu.VMEM((_G, _D), jnp.float32),
        pltpu.VMEM((_G, _D), jnp.float32),
        pltpu.VMEM((_G, _D), jnp.float32),
        pltpu.VMEM((_G, _D), jnp.float32),
        pltpu.VMEM((_NB * 16,), jnp.int32),
        pltpu.SemaphoreType.DMA,
        pltpu.SemaphoreType.DMA,
        pltpu.SemaphoreType.DMA,
        pltpu.SemaphoreType.DMA,
    ],
    compiler_params=pltpu.CompilerParams(needs_layout_passes=False),
)(_hop_body)


def kernel(user_emb, user_offset_emb, item_emb, item_offset_emb, tag_emb,
           tag_offset, graph, W1, b1, W2, b2):
    head = graph[0]
    tail = graph[1]

    all_embs = jnp.concatenate([user_emb, item_emb, tag_emb], axis=0)
    all_off = jnp.concatenate(
        [user_offset_emb, item_offset_emb, tag_offset], axis=0
    )

    bpacked, counts = _partition(head, tail)

    for _ in range(2):
        l2 = _node_mlp(all_embs, W1, b1, W2, b2)
        offa, offb, s, num = _hop(
            l2, all_off, all_embs, bpacked, counts
        )
        all_embs, all_off = _finalize(
            num.reshape(_NP, _D)[:_NN],
            s.reshape(_NP, _D)[:_NN],
            offa.reshape(_NP, _D)[:_NN],
            offb.reshape(_NP, _D)[:_NN],
        )

    return (
        all_embs[:_NU],
        all_off[:_NU],
        all_embs[_NU:_NU + _NI],
        all_off[_NU:_NU + _NI],
        all_embs[_NU + _NI:],
        all_off[_NU + _NI:],
    )


# speedup vs baseline: 1.0273x; 1.0273x over previous
import functools
import jax
import jax.numpy as jnp
from jax import lax
from jax.experimental import pallas as pl
from jax.experimental.pallas import tpu as pltpu
from jax.experimental.pallas import tpu_sc as plsc

_NU, _NI, _NT = 3000, 5000, 2000
_NN = _NU + _NI + _NT
_D = 128
_E = 320000

_NB = 64
_BS = 160
_NP = _NB * _BS
_F = 2048
_ECAP = 158 * _F
_BLK = 2000
_OCAP = 2 * _F
_G = 96
_NG = _G // 16
_NW = 32


def _mlp_body(emb_ref, w1_ref, b1_ref, w2_ref, b2_ref, out_ref):
    h = jnp.maximum(
        jnp.dot(emb_ref[...], w1_ref[...].T, preferred_element_type=jnp.float32)
        + b1_ref[...],
        0.0,
    )
    out_ref[...] = (
        jnp.dot(h, w2_ref[...].T, preferred_element_type=jnp.float32) + b2_ref[...]
    )


def _node_mlp(embs, W1, b1, W2, b2):
    blk = 1000
    return pl.pallas_call(
        _mlp_body,
        grid=(_NN // blk,),
        in_specs=[
            pl.BlockSpec((blk, _D), lambda i: (i, 0)),
            pl.BlockSpec((_D, _D), lambda i: (0, 0)),
            pl.BlockSpec((1, _D), lambda i: (0, 0)),
            pl.BlockSpec((_D, _D), lambda i: (0, 0)),
            pl.BlockSpec((1, _D), lambda i: (0, 0)),
        ],
        out_specs=pl.BlockSpec((blk, _D), lambda i: (i, 0)),
        out_shape=jax.ShapeDtypeStruct((_NN, _D), jnp.float32),
    )(embs, W1, b1.reshape(1, _D), W2, b2.reshape(1, _D))


def _finalize_body(num_ref, s_ref, offa_ref, offb_ref, emb_ref, off_ref):
    i = pl.program_id(0)
    agg = num_ref[...] / (s_ref[...] + 1e-16)
    nrm = jnp.sqrt(jnp.sum(agg * agg, axis=1, keepdims=True))
    emb_ref[...] = agg / jnp.maximum(nrm, 1e-12)
    blk = num_ref.shape[0]
    r = i * blk + lax.broadcasted_iota(jnp.int32, (blk, _D), 0)
    a = offa_ref[...]
    b = offb_ref[...]
    a0 = jnp.where(jnp.isfinite(a), a, 0.0)
    b0 = jnp.where(jnp.isfinite(b), b, 0.0)
    off = jnp.where(
        r < _NU, jnp.minimum(a0, b0), jnp.where(r < _NU + _NI, b0, a0)
    )
    off_ref[...] = jnp.maximum(off, 0.0)


def _finalize(num, s, offa, offb):
    blk = 1000
    return pl.pallas_call(
        _finalize_body,
        grid=(_NN // blk,),
        in_specs=[pl.BlockSpec((blk, _D), lambda i: (i, 0))] * 4,
        out_specs=[pl.BlockSpec((blk, _D), lambda i: (i, 0))] * 2,
        out_shape=[jax.ShapeDtypeStruct((_NN, _D), jnp.float32)] * 2,
    )(num, s, offa, offb)


_sc_mesh = plsc.VectorSubcoreMesh(core_axis_name="c", subcore_axis_name="s")


def _partition_body(head_hbm, tail_hbm, bp_hbm, cnts_hbm,
                    hblk, tblk, hblk2, tblk2, ot0, ot1, cnt16,
                    semA, semB, semf):
    wid = lax.axis_index("s") * 2 + lax.axis_index("c")
    b0 = wid
    b1 = wid + _NW
    lo0 = b0 * _BS
    hi0 = lo0 + _BS
    lo1 = b1 * _BS
    hi1 = lo1 + _BS

    lanes = lax.iota(jnp.int32, 16)
    bsets = ((hblk, tblk, semA), (hblk2, tblk2, semB))
    nblk = _E // _BLK

    def start_blk(i, bs):
        hb, tb, sm = bs
        pltpu.async_copy(
            head_hbm.at[pl.ds(pl.multiple_of(i * _BLK, 8), _BLK)], hb, sm
        )
        pltpu.async_copy(
            tail_hbm.at[pl.ds(pl.multiple_of(i * _BLK, 8), _BLK)], tb, sm
        )

    def wait_blk(bs):
        hb, tb, sm = bs
        pltpu.make_async_copy(head_hbm.at[pl.ds(0, _BLK)], hb, sm).wait()
        pltpu.make_async_copy(tail_hbm.at[pl.ds(0, _BLK)], tb, sm).wait()

    def blk_body(i, carry, hblk, tblk):
        w0, fl0, w1, fl1 = carry

        def vr_body(k, c2):
            w0, w1 = c2
            h = hblk[pl.ds(k * 16, 16)]
            t = tblk[pl.ds(k * 16, 16)]
            m0 = (h >= lo0) & (h < hi0)
            inc0 = plsc.cumsum(m0.astype(jnp.int32))
            pos0 = jnp.where(m0, w0 + inc0 - 1, _OCAP + lanes)
            plsc.store_scatter(ot0, [pos0], t | ((h - lo0) << 16))
            w0 = w0 + inc0[15]
            m1 = (h >= lo1) & (h < hi1)
            inc1 = plsc.cumsum(m1.astype(jnp.int32))
            pos1 = jnp.where(m1, w1 + inc1 - 1, _OCAP + lanes)
            plsc.store_scatter(ot1, [pos1], t | ((h - lo1) << 16))
            w1 = w1 + inc1[15]
            return (w0, w1)

        w0, w1 = lax.fori_loop(0, _BLK // 16, vr_body, (w0, w1))

        do0 = w0 >= _F

        @pl.when(do0)
        def _():
            pltpu.sync_copy(ot0.at[pl.ds(0, _F)],
                            bp_hbm.at[pl.ds(pl.multiple_of(b0 * _ECAP + fl0, 8), _F)])
            for k in range(_BLK // 16):
                ot0[pl.ds(k * 16, 16)] = ot0[pl.ds(_F + k * 16, 16)]

        w0 = jnp.where(do0, w0 - _F, w0)
        fl0 = jnp.where(do0, fl0 + _F, fl0)

        do1 = w1 >= _F

        @pl.when(do1)
        def _():
            pltpu.sync_copy(ot1.at[pl.ds(0, _F)],
                            bp_hbm.at[pl.ds(pl.multiple_of(b1 * _ECAP + fl1, 8), _F)])
            for k in range(_BLK // 16):
                ot1[pl.ds(k * 16, 16)] = ot1[pl.ds(_F + k * 16, 16)]

        w1 = jnp.where(do1, w1 - _F, w1)
        fl1 = jnp.where(do1, fl1 + _F, fl1)
        return (w0, fl0, w1, fl1)

    z = jnp.int32(0)
    for s in range(2):
        start_blk(jnp.int32(s), bsets[s])

    def pair(p, carry):
        for s in range(2):
            i = p * 2 + s
            bs = bsets[s]
            wait_blk(bs)
            carry = blk_body(i, carry, bs[0], bs[1])

            @pl.when(i + 2 < nblk)
            def _(i=i, bs=bs):
                start_blk(i + 2, bs)

        return carry

    w0, fl0, w1, fl1 = lax.fori_loop(0, nblk // 2, pair, (z, z, z, z))

    @pl.when(w0 > 0)
    def _():
        pltpu.sync_copy(ot0.at[pl.ds(0, _F)],
                        bp_hbm.at[pl.ds(pl.multiple_of(b0 * _ECAP + fl0, 8), _F)])

    cnt16[...] = jnp.full((16,), w0 + fl0, jnp.int32)
    pltpu.sync_copy(cnt16, cnts_hbm.at[pl.ds(pl.multiple_of(b0 * 16, 8), 16)])

    @pl.when(w1 > 0)
    def _():
        pltpu.sync_copy(ot1.at[pl.ds(0, _F)],
                        bp_hbm.at[pl.ds(pl.multiple_of(b1 * _ECAP + fl1, 8), _F)])

    cnt16[...] = jnp.full((16,), w1 + fl1, jnp.int32)
    pltpu.sync_copy(cnt16, cnts_hbm.at[pl.ds(pl.multiple_of(b1 * 16, 8), 16)])


_partition = functools.partial(
    pl.kernel,
    out_type=[
        jax.ShapeDtypeStruct((_NB * _ECAP,), jnp.int32),
        jax.ShapeDtypeStruct((_NB * 16,), jnp.int32),
    ],
    mesh=_sc_mesh,
    scratch_types=[
        pltpu.VMEM((_BLK,), jnp.int32),
        pltpu.VMEM((_BLK,), jnp.int32),
        pltpu.VMEM((_BLK,), jnp.int32),
        pltpu.VMEM((_BLK,), jnp.int32),
        pltpu.VMEM((_OCAP + 16,), jnp.int32),
        pltpu.VMEM((_OCAP + 16,), jnp.int32),
        pltpu.VMEM((16,), jnp.int32),
        pltpu.SemaphoreType.DMA,
        pltpu.SemaphoreType.DMA,
        pltpu.SemaphoreType.DMA,
    ],
    compiler_params=pltpu.CompilerParams(needs_layout_passes=False),
)(_partition_body)


def _hop_body(l2_hbm, off_hbm, emb_hbm, bp_hbm, cnts_hbm,
              offa_hbm, offb_hbm, s_hbm, num_hbm,
              acc0, acc1, acc2,
              pk0, pk1, ti0, ti1, hb0, hb1, r0a, r0b, r1a, r1b, counts_v,
              sem0, sem1, semi0, semi1):
    wid = lax.axis_index("s") * 2 + lax.axis_index("c")
    pltpu.sync_copy(cnts_hbm, counts_v)

    ninf = jnp.full((16,), -jnp.inf, jnp.float32)
    pinf = jnp.full((16,), jnp.inf, jnp.float32)
    zero = jnp.zeros((16,), jnp.float32)
    sets = (
        (pk0, ti0, hb0, r0a, r1a, sem0, semi0),
        (pk1, ti1, hb1, r0b, r1b, sem1, semi1),
    )

    def run_pass(bsafe, cnt, other_hbm, group_fn):
        nch = lax.div(cnt + (_G - 1), _G)

        def start_idx(c, st):
            pk = st[0]
            base = pl.multiple_of(bsafe * _ECAP + c * _G, 8)
            pltpu.async_copy(bp_hbm.at[pl.ds(base, _G)], pk, st[6])

        def wait_idx(st):
            pltpu.make_async_copy(bp_hbm.at[pl.ds(0, _G)], st[0], st[6]).wait()

        def stage_rows(st):
            pk, ti, hb, rr0, rr1, sm, smi = st
            for k in range(_NG):
                pv = pk[pl.ds(k * 16, 16)]
                ti[pl.ds(k * 16, 16)] = jnp.clip(pv & 0xFFFF, 0, _NN - 1)
                hb[pl.ds(k * 16, 16)] = jnp.clip(
                    lax.shift_right_arithmetic(pv, 16), 0, _BS - 1
                )
            pltpu.async_copy(l2_hbm.at[ti], rr0, sm)
            pltpu.async_copy(other_hbm.at[ti], rr1, sm)

        def wait_rows(st):
            pk, ti, hb, rr0, rr1, sm, smi = st
            pltpu.make_async_copy(l2_hbm.at[ti], rr0, sm).wait()
            pltpu.make_async_copy(other_hbm.at[ti], rr1, sm).wait()

        for s in range(2):
            @pl.when(s < nch)
            def _(s=s):
                st = sets[s]
                start_idx(jnp.int32(s), st)
                wait_idx(st)
                stage_rows(st)

                @pl.when(s + 2 < nch)
                def _(s=s, st=st):
                    start_idx(jnp.int32(s + 2), st)

        def pair(p, _):
            for s in range(2):
                c = p * 2 + s
                st = sets[s]

                @pl.when(c < nch)
                def _(c=c, st=st):
                    wait_rows(st)
                    nj = jnp.minimum(_G, cnt - c * _G)
                    group_fn(st, nj)

                    @pl.when(c + 2 < nch)
                    def _(c=c, st=st):
                        wait_idx(st)
                        stage_rows(st)

                        @pl.when(c + 4 < nch)
                        def _(c=c, st=st):
                            start_idx(c + 4, st)

            return 0

        lax.fori_loop(0, (nch + 1) // 2, pair, 0)

    def bucket_body(it, _):
        b = wid + it * _NW
        nb = b * _BS
        cnt = counts_v[pl.ds(b * 16, 16)][0]

        def init_a(k, _):
            acc0[pl.ds(k * 16, 16)] = ninf
            acc1[pl.ds(k * 16, 16)] = pinf
            acc2[pl.ds(k * 16, 16)] = ninf
            return 0

        lax.fori_loop(0, _BS * _D // 16, init_a, 0)

        def group_a(st, nj):
            pk, ti, hb, rr0, rr1, sm, smi = st

            def g_body(g, _):
                hv = hb[pl.ds(g * 16, 16)]
                tv = ti[pl.ds(g * 16, 16)]
                hs = [hv[l] for l in range(16)]
                ts = [tv[l] for l in range(16)]
                for l in range(16):
                    h = hs[l]
                    t = ts[l]
                    j = g * 16 + l
                    valid = j < nj
                    hg = h + nb
                    ca = ((hg < _NU) & (t >= _NU) & (t < _NU + _NI)) | (
                        hg >= _NU + _NI
                    )
                    cb = ((hg < _NU) & (t >= _NU + _NI)) | (
                        (hg >= _NU) & (hg < _NU + _NI)
                    )
                    sa = ca & valid
                    sb = cb & valid
                    rbase = h * _D
                    ld = []
                    for cc in range(8):
                        o = cc * 16
                        ld.append((
                            o,
                            rr0[j, pl.ds(o, 16)],
                            rr1[j, pl.ds(o, 16)],
                            acc0[pl.ds(rbase + o, 16)],
                            acc1[pl.ds(rbase + o, 16)],
                            acc2[pl.ds(rbase + o, 16)],
                        ))
                    for o, r0v, r1v, a0v, a1v, a2v in ld:
                        r1r = jnp.maximum(r1v, 0.0)
                        acc0[pl.ds(rbase + o, 16)] = jnp.maximum(
                            a0v, jnp.where(valid, r0v, ninf)
                        )
                        acc1[pl.ds(rbase + o, 16)] = jnp.minimum(
                            a1v, jnp.where(sa, r1r, pinf)
                        )
                        acc2[pl.ds(rbase + o, 16)] = jnp.maximum(
                            a2v, jnp.where(sb, r1r, ninf)
                        )
                return 0

            lax.fori_loop(0, _NG, g_body, 0)

        run_pass(b, cnt, off_hbm, group_a)

        pltpu.sync_copy(acc1, offa_hbm.at[pl.ds(pl.multiple_of(nb * _D, 8), _BS * _D)])
        pltpu.sync_copy(acc2, offb_hbm.at[pl.ds(pl.multiple_of(nb * _D, 8), _BS * _D)])

        def init_b(k, _):
            acc1[pl.ds(k * 16, 16)] = zero
            acc2[pl.ds(k * 16, 16)] = zero
            return 0

        lax.fori_loop(0, _BS * _D // 16, init_b, 0)

        def group_b(st, nj):
            pk, ti, hb, rr0, rr1, sm, smi = st

            def g_body(g, _):
                hv = hb[pl.ds(g * 16, 16)]
                hs = [hv[l] for l in range(16)]
                for l in range(16):
                    h = hs[l]
                    j = g * 16 + l
                    valid = j < nj
                    rbase = h * _D
                    ld = []
                    for cc in range(8):
                        o = cc * 16
                        ld.append((
                            o,
                            rr0[j, pl.ds(o, 16)],
                            rr1[j, pl.ds(o, 16)],
                            acc0[pl.ds(rbase + o, 16)],
                            acc1[pl.ds(rbase + o, 16)],
                            acc2[pl.ds(rbase + o, 16)],
                        ))
                    for o, l2v, embv, mv, s1v, n1v in ld:
                        ev = jnp.where(valid, jnp.exp(l2v - mv), zero)
                        acc1[pl.ds(rbase + o, 16)] = s1v + ev
                        acc2[pl.ds(rbase + o, 16)] = n1v + ev * embv
                return 0

            lax.fori_loop(0, _NG, g_body, 0)

        run_pass(b, cnt, emb_hbm, group_b)

        pltpu.sync_copy(acc1, s_hbm.at[pl.ds(pl.multiple_of(nb * _D, 8), _BS * _D)])
        pltpu.sync_copy(acc2, num_hbm.at[pl.ds(pl.multiple_of(nb * _D, 8), _BS * _D)])
        return 0

    lax.fori_loop(0, 2, bucket_body, 0)


_hop = functools.partial(
    pl.kernel,
    out_type=[jax.ShapeDtypeStruct((_NP * _D,), jnp.float32)] * 4,
    mesh=_sc_mesh,
    scratch_types=[
        pltpu.VMEM((_BS * _D,), jnp.float32),
        pltpu.VMEM((_BS * _D,), jnp.float32),
        pltpu.VMEM((_BS * _D,), jnp.float32),
        pltpu.VMEM((_G,), jnp.int32),
        pltpu.VMEM((_G,), jnp.int32),
        pltpu.VMEM((_G,), jnp.int32),
        pltpu.VMEM((_G,), jnp.int32),
        pltpu.VMEM((_G,), jnp.int32),
        pltpu.VMEM((_G,), jnp.int32),
        pltpu.VMEM((_G, _D), jnp.float32),
        pltpu.VMEM((_G, _D), jnp.float32),
        pltpu.VMEM((_G, _D), jnp.float32),
        pltpu.VMEM((_G, _D), jnp.float32),
        pltpu.VMEM((_NB * 16,), jnp.int32),
        pltpu.SemaphoreType.DMA,
        pltpu.SemaphoreType.DMA,
        pltpu.SemaphoreType.DMA,
        pltpu.SemaphoreType.DMA,
    ],
    compiler_params=pltpu.CompilerParams(needs_layout_passes=False),
)(_hop_body)


def kernel(user_emb, user_offset_emb, item_emb, item_offset_emb, tag_emb,
           tag_offset, graph, W1, b1, W2, b2):
    head = graph[0]
    tail = graph[1]

    all_embs = jnp.concatenate([user_emb, item_emb, tag_emb], axis=0)
    all_off = jnp.concatenate(
        [user_offset_emb, item_offset_emb, tag_offset], axis=0
    )

    bpacked, counts = _partition(head, tail)

    for _ in range(2):
        l2 = _node_mlp(all_embs, W1, b1, W2, b2)
        offa, offb, s, num = _hop(
            l2, all_off, all_embs, bpacked, counts
        )
        all_embs, all_off = _finalize(
            num.reshape(_NP, _D)[:_NN],
            s.reshape(_NP, _D)[:_NN],
            offa.reshape(_NP, _D)[:_NN],
            offb.reshape(_NP, _D)[:_NN],
        )

    return (
        all_embs[:_NU],
        all_off[:_NU],
        all_embs[_NU:_NU + _NI],
        all_off[_NU:_NU + _NI],
        all_embs[_NU + _NI:],
        all_off[_NU + _NI:],
    )


# speedup vs baseline: 1.0380x; 1.0104x over previous
import functools
import jax
import jax.numpy as jnp
from jax import lax
from jax.experimental import pallas as pl
from jax.experimental.pallas import tpu as pltpu
from jax.experimental.pallas import tpu_sc as plsc

_NU, _NI, _NT = 3000, 5000, 2000
_NN = _NU + _NI + _NT
_D = 128
_E = 320000

_NB = 64
_BS = 160
_NP = _NB * _BS
_F = 2048
_ECAP = 158 * _F
_BLK = 2000
_OCAP = 2 * _F
_G = 96
_NG = _G // 16
_NW = 32


def _mlp_body(emb_ref, w1_ref, b1_ref, w2_ref, b2_ref, out_ref):
    h = jnp.maximum(
        jnp.dot(emb_ref[...], w1_ref[...].T, preferred_element_type=jnp.float32)
        + b1_ref[...],
        0.0,
    )
    out_ref[...] = (
        jnp.dot(h, w2_ref[...].T, preferred_element_type=jnp.float32) + b2_ref[...]
    )


def _node_mlp(embs, W1, b1, W2, b2):
    blk = 1000
    return pl.pallas_call(
        _mlp_body,
        grid=(_NN // blk,),
        in_specs=[
            pl.BlockSpec((blk, _D), lambda i: (i, 0)),
            pl.BlockSpec((_D, _D), lambda i: (0, 0)),
            pl.BlockSpec((1, _D), lambda i: (0, 0)),
            pl.BlockSpec((_D, _D), lambda i: (0, 0)),
            pl.BlockSpec((1, _D), lambda i: (0, 0)),
        ],
        out_specs=pl.BlockSpec((blk, _D), lambda i: (i, 0)),
        out_shape=jax.ShapeDtypeStruct((_NN, _D), jnp.float32),
    )(embs, W1, b1.reshape(1, _D), W2, b2.reshape(1, _D))


def _finalize_body(num_ref, s_ref, offa_ref, offb_ref, emb_ref, off_ref):
    i = pl.program_id(0)
    agg = num_ref[...] / (s_ref[...] + 1e-16)
    nrm = jnp.sqrt(jnp.sum(agg * agg, axis=1, keepdims=True))
    emb_ref[...] = agg / jnp.maximum(nrm, 1e-12)
    blk = num_ref.shape[0]
    r = i * blk + lax.broadcasted_iota(jnp.int32, (blk, _D), 0)
    a = offa_ref[...]
    b = offb_ref[...]
    a0 = jnp.where(jnp.isfinite(a), a, 0.0)
    b0 = jnp.where(jnp.isfinite(b), -b, 0.0)
    off = jnp.where(
        r < _NU, jnp.minimum(a0, b0), jnp.where(r < _NU + _NI, b0, a0)
    )
    off_ref[...] = jnp.maximum(off, 0.0)


def _finalize(num, s, offa, offb):
    blk = 1000
    return pl.pallas_call(
        _finalize_body,
        grid=(_NN // blk,),
        in_specs=[pl.BlockSpec((blk, _D), lambda i: (i, 0))] * 4,
        out_specs=[pl.BlockSpec((blk, _D), lambda i: (i, 0))] * 2,
        out_shape=[jax.ShapeDtypeStruct((_NN, _D), jnp.float32)] * 2,
    )(num, s, offa, offb)


_sc_mesh = plsc.VectorSubcoreMesh(core_axis_name="c", subcore_axis_name="s")


def _partition_body(head_hbm, tail_hbm, bp_hbm, cnts_hbm,
                    hblk, tblk, hblk2, tblk2, ot0, ot1, cnt16,
                    semA, semB, semf):
    wid = lax.axis_index("s") * 2 + lax.axis_index("c")
    b0 = wid
    b1 = wid + _NW
    lo0 = b0 * _BS
    hi0 = lo0 + _BS
    lo1 = b1 * _BS
    hi1 = lo1 + _BS

    lanes = lax.iota(jnp.int32, 16)
    bsets = ((hblk, tblk, semA), (hblk2, tblk2, semB))
    nblk = _E // _BLK

    def start_blk(i, bs):
        hb, tb, sm = bs
        pltpu.async_copy(
            head_hbm.at[pl.ds(pl.multiple_of(i * _BLK, 8), _BLK)], hb, sm
        )
        pltpu.async_copy(
            tail_hbm.at[pl.ds(pl.multiple_of(i * _BLK, 8), _BLK)], tb, sm
        )

    def wait_blk(bs):
        hb, tb, sm = bs
        pltpu.make_async_copy(head_hbm.at[pl.ds(0, _BLK)], hb, sm).wait()
        pltpu.make_async_copy(tail_hbm.at[pl.ds(0, _BLK)], tb, sm).wait()

    def blk_body(i, carry, hblk, tblk):
        w0, fl0, w1, fl1 = carry

        def vr_body(k, c2):
            w0, w1 = c2
            h = hblk[pl.ds(k * 16, 16)]
            t = tblk[pl.ds(k * 16, 16)]
            m0 = (h >= lo0) & (h < hi0)
            inc0 = plsc.cumsum(m0.astype(jnp.int32))
            pos0 = jnp.where(m0, w0 + inc0 - 1, _OCAP + lanes)
            plsc.store_scatter(ot0, [pos0], t | ((h - lo0) << 16))
            w0 = w0 + inc0[15]
            m1 = (h >= lo1) & (h < hi1)
            inc1 = plsc.cumsum(m1.astype(jnp.int32))
            pos1 = jnp.where(m1, w1 + inc1 - 1, _OCAP + lanes)
            plsc.store_scatter(ot1, [pos1], t | ((h - lo1) << 16))
            w1 = w1 + inc1[15]
            return (w0, w1)

        w0, w1 = lax.fori_loop(0, _BLK // 16, vr_body, (w0, w1))

        do0 = w0 >= _F

        @pl.when(do0)
        def _():
            pltpu.sync_copy(ot0.at[pl.ds(0, _F)],
                            bp_hbm.at[pl.ds(pl.multiple_of(b0 * _ECAP + fl0, 8), _F)])
            for k in range(_BLK // 16):
                ot0[pl.ds(k * 16, 16)] = ot0[pl.ds(_F + k * 16, 16)]

        w0 = jnp.where(do0, w0 - _F, w0)
        fl0 = jnp.where(do0, fl0 + _F, fl0)

        do1 = w1 >= _F

        @pl.when(do1)
        def _():
            pltpu.sync_copy(ot1.at[pl.ds(0, _F)],
                            bp_hbm.at[pl.ds(pl.multiple_of(b1 * _ECAP + fl1, 8), _F)])
            for k in range(_BLK // 16):
                ot1[pl.ds(k * 16, 16)] = ot1[pl.ds(_F + k * 16, 16)]

        w1 = jnp.where(do1, w1 - _F, w1)
        fl1 = jnp.where(do1, fl1 + _F, fl1)
        return (w0, fl0, w1, fl1)

    z = jnp.int32(0)
    for s in range(2):
        start_blk(jnp.int32(s), bsets[s])

    def pair(p, carry):
        for s in range(2):
            i = p * 2 + s
            bs = bsets[s]
            wait_blk(bs)
            carry = blk_body(i, carry, bs[0], bs[1])

            @pl.when(i + 2 < nblk)
            def _(i=i, bs=bs):
                start_blk(i + 2, bs)

        return carry

    w0, fl0, w1, fl1 = lax.fori_loop(0, nblk // 2, pair, (z, z, z, z))

    @pl.when(w0 > 0)
    def _():
        pltpu.sync_copy(ot0.at[pl.ds(0, _F)],
                        bp_hbm.at[pl.ds(pl.multiple_of(b0 * _ECAP + fl0, 8), _F)])

    cnt16[...] = jnp.full((16,), w0 + fl0, jnp.int32)
    pltpu.sync_copy(cnt16, cnts_hbm.at[pl.ds(pl.multiple_of(b0 * 16, 8), 16)])

    @pl.when(w1 > 0)
    def _():
        pltpu.sync_copy(ot1.at[pl.ds(0, _F)],
                        bp_hbm.at[pl.ds(pl.multiple_of(b1 * _ECAP + fl1, 8), _F)])

    cnt16[...] = jnp.full((16,), w1 + fl1, jnp.int32)
    pltpu.sync_copy(cnt16, cnts_hbm.at[pl.ds(pl.multiple_of(b1 * 16, 8), 16)])


_partition = functools.partial(
    pl.kernel,
    out_type=[
        jax.ShapeDtypeStruct((_NB * _ECAP,), jnp.int32),
        jax.ShapeDtypeStruct((_NB * 16,), jnp.int32),
    ],
    mesh=_sc_mesh,
    scratch_types=[
        pltpu.VMEM((_BLK,), jnp.int32),
        pltpu.VMEM((_BLK,), jnp.int32),
        pltpu.VMEM((_BLK,), jnp.int32),
        pltpu.VMEM((_BLK,), jnp.int32),
        pltpu.VMEM((_OCAP + 16,), jnp.int32),
        pltpu.VMEM((_OCAP + 16,), jnp.int32),
        pltpu.VMEM((16,), jnp.int32),
        pltpu.SemaphoreType.DMA,
        pltpu.SemaphoreType.DMA,
        pltpu.SemaphoreType.DMA,
    ],
    compiler_params=pltpu.CompilerParams(needs_layout_passes=False),
)(_partition_body)


def _hop_body(l2_hbm, off_hbm, emb_hbm, bp_hbm, cnts_hbm,
              offa_hbm, offb_hbm, s_hbm, num_hbm,
              acc0, accO,
              pk0, pk1, ti0, ti1, hb0, hb1, r0a, r0b, r1a, r1b, counts_v,
              sem0, sem1, semi0, semi1):
    wid = lax.axis_index("s") * 2 + lax.axis_index("c")
    pltpu.sync_copy(cnts_hbm, counts_v)

    ninf = jnp.full((16,), -jnp.inf, jnp.float32)
    pinf = jnp.full((16,), jnp.inf, jnp.float32)
    zero = jnp.zeros((16,), jnp.float32)
    sets = (
        (pk0, ti0, hb0, r0a, r1a, sem0, semi0),
        (pk1, ti1, hb1, r0b, r1b, sem1, semi1),
    )

    def run_pass(bsafe, cnt, other_hbm, group_fn):
        nch = lax.div(cnt + (_G - 1), _G)

        def start_idx(c, st):
            pk = st[0]
            base = pl.multiple_of(bsafe * _ECAP + c * _G, 8)
            pltpu.async_copy(bp_hbm.at[pl.ds(base, _G)], pk, st[6])

        def wait_idx(st):
            pltpu.make_async_copy(bp_hbm.at[pl.ds(0, _G)], st[0], st[6]).wait()

        def stage_rows(st):
            pk, ti, hb, rr0, rr1, sm, smi = st
            for k in range(_NG):
                pv = pk[pl.ds(k * 16, 16)]
                ti[pl.ds(k * 16, 16)] = jnp.clip(pv & 0xFFFF, 0, _NN - 1)
                hb[pl.ds(k * 16, 16)] = jnp.clip(
                    lax.shift_right_arithmetic(pv, 16), 0, _BS - 1
                )
            pltpu.async_copy(l2_hbm.at[ti], rr0, sm)
            pltpu.async_copy(other_hbm.at[ti], rr1, sm)

        def wait_rows(st):
            pk, ti, hb, rr0, rr1, sm, smi = st
            pltpu.make_async_copy(l2_hbm.at[ti], rr0, sm).wait()
            pltpu.make_async_copy(other_hbm.at[ti], rr1, sm).wait()

        for s in range(2):
            @pl.when(s < nch)
            def _(s=s):
                st = sets[s]
                start_idx(jnp.int32(s), st)
                wait_idx(st)
                stage_rows(st)

                @pl.when(s + 2 < nch)
                def _(s=s, st=st):
                    start_idx(jnp.int32(s + 2), st)

        def pair(p, _):
            for s in range(2):
                c = p * 2 + s
                st = sets[s]

                @pl.when(c < nch)
                def _(c=c, st=st):
                    wait_rows(st)
                    nj = jnp.minimum(_G, cnt - c * _G)
                    group_fn(st, nj)

                    @pl.when(c + 2 < nch)
                    def _(c=c, st=st):
                        wait_idx(st)
                        stage_rows(st)

                        @pl.when(c + 4 < nch)
                        def _(c=c, st=st):
                            start_idx(c + 4, st)

            return 0

        lax.fori_loop(0, (nch + 1) // 2, pair, 0)

    def bucket_body(it, _):
        b = wid + it * _NW
        nb = b * _BS
        cnt = counts_v[pl.ds(b * 16, 16)][0]

        def init_a(k, _):
            acc0[pl.ds(k * 16, 16)] = ninf
            accO[pl.ds(k * 16, 16)] = pinf
            accO[pl.ds(_BS * _D + k * 16, 16)] = pinf
            return 0

        lax.fori_loop(0, _BS * _D // 16, init_a, 0)

        def group_a(st, nj):
            pk, ti, hb, rr0, rr1, sm, smi = st

            def g_body(g, _):
                hv = hb[pl.ds(g * 16, 16)]
                tv = ti[pl.ds(g * 16, 16)]
                hs = [hv[l] for l in range(16)]
                ts = [tv[l] for l in range(16)]
                for l in range(16):
                    h = hs[l]
                    t = ts[l]
                    j = g * 16 + l
                    valid = j < nj
                    hg = h + nb
                    ca = ((hg < _NU) & (t >= _NU) & (t < _NU + _NI)) | (
                        hg >= _NU + _NI
                    )
                    cb = ((hg < _NU) & (t >= _NU + _NI)) | (
                        (hg >= _NU) & (hg < _NU + _NI)
                    )
                    s_ok = (ca | cb) & valid
                    rbase = h * _D
                    obase = jnp.where(ca, 0, _BS * _D) + rbase
                    ld = []
                    for cc in range(8):
                        o = cc * 16
                        ld.append((
                            o,
                            rr0[j, pl.ds(o, 16)],
                            rr1[j, pl.ds(o, 16)],
                            acc0[pl.ds(rbase + o, 16)],
                            accO[pl.ds(obase + o, 16)],
                        ))
                    for o, r0v, r1v, a0v, aov in ld:
                        r1r = jnp.maximum(r1v, 0.0)
                        val = jnp.where(ca, r1r, -r1r)
                        acc0[pl.ds(rbase + o, 16)] = jnp.maximum(
                            a0v, jnp.where(valid, r0v, ninf)
                        )
                        accO[pl.ds(obase + o, 16)] = jnp.minimum(
                            aov, jnp.where(s_ok, val, pinf)
                        )
                return 0

            lax.fori_loop(0, _NG, g_body, 0)

        run_pass(b, cnt, off_hbm, group_a)

        pltpu.sync_copy(accO.at[pl.ds(0, _BS * _D)],
                        offa_hbm.at[pl.ds(pl.multiple_of(nb * _D, 8), _BS * _D)])
        pltpu.sync_copy(accO.at[pl.ds(_BS * _D, _BS * _D)],
                        offb_hbm.at[pl.ds(pl.multiple_of(nb * _D, 8), _BS * _D)])

        def init_b(k, _):
            accO[pl.ds(k * 16, 16)] = zero
            accO[pl.ds(_BS * _D + k * 16, 16)] = zero
            return 0

        lax.fori_loop(0, _BS * _D // 16, init_b, 0)

        def group_b(st, nj):
            pk, ti, hb, rr0, rr1, sm, smi = st

            def g_body(g, _):
                hv = hb[pl.ds(g * 16, 16)]
                hs = [hv[l] for l in range(16)]
                for l in range(16):
                    h = hs[l]
                    j = g * 16 + l
                    valid = j < nj
                    rbase = h * _D
                    ld = []
                    for cc in range(8):
                        o = cc * 16
                        ld.append((
                            o,
                            rr0[j, pl.ds(o, 16)],
                            rr1[j, pl.ds(o, 16)],
                            acc0[pl.ds(rbase + o, 16)],
                            accO[pl.ds(rbase + o, 16)],
                            accO[pl.ds(_BS * _D + rbase + o, 16)],
                        ))
                    for o, l2v, embv, mv, s1v, n1v in ld:
                        ev = jnp.where(valid, jnp.exp(l2v - mv), zero)
                        accO[pl.ds(rbase + o, 16)] = s1v + ev
                        accO[pl.ds(_BS * _D + rbase + o, 16)] = n1v + ev * embv
                return 0

            lax.fori_loop(0, _NG, g_body, 0)

        run_pass(b, cnt, emb_hbm, group_b)

        pltpu.sync_copy(accO.at[pl.ds(0, _BS * _D)],
                        s_hbm.at[pl.ds(pl.multiple_of(nb * _D, 8), _BS * _D)])
        pltpu.sync_copy(accO.at[pl.ds(_BS * _D, _BS * _D)],
                        num_hbm.at[pl.ds(pl.multiple_of(nb * _D, 8), _BS * _D)])
        return 0

    lax.fori_loop(0, 2, bucket_body, 0)


_hop = functools.partial(
    pl.kernel,
    out_type=[jax.ShapeDtypeStruct((_NP * _D,), jnp.float32)] * 4,
    mesh=_sc_mesh,
    scratch_types=[
        pltpu.VMEM((_BS * _D,), jnp.float32),
        pltpu.VMEM((2 * _BS * _D,), jnp.float32),
        pltpu.VMEM((_G,), jnp.int32),
        pltpu.VMEM((_G,), jnp.int32),
        pltpu.VMEM((_G,), jnp.int32),
        pltpu.VMEM((_G,), jnp.int32),
        pltpu.VMEM((_G,), jnp.int32),
        pltpu.VMEM((_G,), jnp.int32),
        pltpu.VMEM((_G, _D), jnp.float32),
        pltpu.VMEM((_G, _D), jnp.float32),
        pltpu.VMEM((_G, _D), jnp.float32),
        pltpu.VMEM((_G, _D), jnp.float32),
        pltpu.VMEM((_NB * 16,), jnp.int32),
        pltpu.SemaphoreType.DMA,
        pltpu.SemaphoreType.DMA,
        pltpu.SemaphoreType.DMA,
        pltpu.SemaphoreType.DMA,
    ],
    compiler_params=pltpu.CompilerParams(needs_layout_passes=False),
)(_hop_body)


def kernel(user_emb, user_offset_emb, item_emb, item_offset_emb, tag_emb,
           tag_offset, graph, W1, b1, W2, b2):
    head = graph[0]
    tail = graph[1]

    all_embs = jnp.concatenate([user_emb, item_emb, tag_emb], axis=0)
    all_off = jnp.concatenate(
        [user_offset_emb, item_offset_emb, tag_offset], axis=0
    )

    bpacked, counts = _partition(head, tail)

    for _ in range(2):
        l2 = _node_mlp(all_embs, W1, b1, W2, b2)
        offa, offb, s, num = _hop(
            l2, all_off, all_embs, bpacked, counts
        )
        all_embs, all_off = _finalize(
            num.reshape(_NP, _D)[:_NN],
            s.reshape(_NP, _D)[:_NN],
            offa.reshape(_NP, _D)[:_NN],
            offb.reshape(_NP, _D)[:_NN],
        )

    return (
        all_embs[:_NU],
        all_off[:_NU],
        all_embs[_NU:_NU + _NI],
        all_off[_NU:_NU + _NI],
        all_embs[_NU + _NI:],
        all_off[_NU + _NI:],
    )
